# baseline (speedup 1.0000x reference)
import functools

import jax
import jax.numpy as jnp
from jax import lax
from jax.experimental import pallas as pl
from jax.experimental.pallas import tpu as pltpu
from jax.experimental.pallas import tpu_sc as plsc

N = 10000
NPAD = 10240
NSCR = NPAD - N
D = 64
AUG = 72
DG = 8
NC = 2
NS = 16
NW = NC * NS
CHUNK = 128
E = 320000
E2 = 2 * E
EPW = E2 // NW
NCH = 158
WPE = NCH * CHUNK
RPT = NPAD // NS
RPTL = N // NS

_mesh = plsc.VectorSubcoreMesh(core_axis_name="c", subcore_axis_name="s")
_sc_params = pltpu.CompilerParams(use_tc_tiling_on_sc=False,
                                  needs_layout_passes=False)


def _round_body(s_hbm, src_hbm, dst_hbm, zcst_hbm, out_hbm,
                idx_s, idx_d, b0, b1, acc, sg0, sg1, ss0, ss1):
    cid = lax.axis_index("c")
    sid = lax.axis_index("s")
    wid = sid * NC + cid
    r0 = sid * RPT

    for k in range(RPT // CHUNK):
        pltpu.sync_copy(zcst_hbm, acc.at[pl.ds(r0 + k * CHUNK, CHUNK)])
    pltpu.sync_copy(src_hbm.at[wid], idx_s.at[pl.ds(0, NCH)])
    pltpu.sync_copy(dst_hbm.at[wid], idx_d)
    zi = jnp.zeros((16,), jnp.int32)
    for k in range(CHUNK // 16):
        idx_s[NCH, pl.ds(k * 16, 16)] = zi
    plsc.subcore_barrier()

    pltpu.async_copy(s_hbm.at[idx_s.at[0]], b0, sg0)

    def pair(i, _):
        j = 2 * i
        pltpu.async_copy(s_hbm.at[idx_s.at[j + 1]], b1, sg1)
        pltpu.make_async_copy(s_hbm.at[idx_s.at[j]], b0, sg0).wait()
        sd0 = pltpu.async_copy(b0, acc.at[idx_d.at[j]], ss0, add=True)
        pltpu.make_async_copy(s_hbm.at[idx_s.at[j + 1]], b1, sg1).wait()
        sd1 = pltpu.async_copy(b1, acc.at[idx_d.at[j + 1]], ss1, add=True)
        sd0.wait()
        pltpu.async_copy(s_hbm.at[idx_s.at[j + 2]], b0, sg0)
        sd1.wait()
        return 0

    lax.fori_loop(0, NCH // 2, pair, 0)
    pltpu.make_async_copy(s_hbm.at[idx_s.at[NCH]], b0, sg0).wait()
    plsc.subcore_barrier()
    pltpu.sync_copy(acc.at[pl.ds(r0, RPT)], out_hbm.at[cid, pl.ds(r0, RPT)])


_round = pl.kernel(
    _round_body,
    out_type=jax.ShapeDtypeStruct((NC, NPAD, AUG), jnp.float32),
    mesh=_mesh,
    compiler_params=_sc_params,
    scratch_types=[
        pltpu.VMEM((NCH + 1, CHUNK), jnp.int32),
        pltpu.VMEM((NCH, CHUNK), jnp.int32),
        pltpu.VMEM((CHUNK, AUG), jnp.float32),
        pltpu.VMEM((CHUNK, AUG), jnp.float32),
        pltpu.VMEM_SHARED((NPAD, AUG), jnp.float32),
        pltpu.SemaphoreType.DMA,
        pltpu.SemaphoreType.DMA,
        pltpu.SemaphoreType.DMA,
        pltpu.SemaphoreType.DMA,
    ],
)


def _deg_body(dst_hbm, cst_hbm, out_hbm, idx_d, ubuf, acc, ss0, ss1):
    cid = lax.axis_index("c")
    sid = lax.axis_index("s")
    wid = sid * NC + cid
    r0 = sid * RPT

    pltpu.sync_copy(cst_hbm, ubuf)
    for k in range(RPT // CHUNK):
        pltpu.sync_copy(ubuf.at[pl.ds(CHUNK, CHUNK)],
                        acc.at[pl.ds(r0 + k * CHUNK, CHUNK)])
    pltpu.sync_copy(dst_hbm.at[wid], idx_d)
    plsc.subcore_barrier()

    def pair(i, _):
        j = 2 * i
        d0 = pltpu.async_copy(ubuf.at[pl.ds(0, CHUNK)],
                              acc.at[idx_d.at[j]], ss0, add=True)
        d1 = pltpu.async_copy(ubuf.at[pl.ds(0, CHUNK)],
                              acc.at[idx_d.at[j + 1]], ss1, add=True)
        d0.wait()
        d1.wait()
        return 0

    lax.fori_loop(0, NCH // 2, pair, 0)
    plsc.subcore_barrier()
    pltpu.sync_copy(acc.at[pl.ds(r0, RPT)], out_hbm.at[cid, pl.ds(r0, RPT)])


_deg_round = pl.kernel(
    _deg_body,
    out_type=jax.ShapeDtypeStruct((NC, NPAD, DG), jnp.float32),
    mesh=_mesh,
    compiler_params=_sc_params,
    scratch_types=[
        pltpu.VMEM((NCH, CHUNK), jnp.int32),
        pltpu.VMEM((2 * CHUNK, DG), jnp.float32),
        pltpu.VMEM_SHARED((NPAD, DG), jnp.float32),
        pltpu.SemaphoreType.DMA,
        pltpu.SemaphoreType.DMA,
    ],
)


def _logits_body(z_hbm, i0_hbm, i1_hbm, out_hbm,
                 idx0, idx1, a0, a1, b0, b1, lbuf, fbuf,
                 sa0, sa1, sb0, sb1):
    cid = lax.axis_index("c")
    sid = lax.axis_index("s")
    wid = sid * NC + cid

    pltpu.sync_copy(i0_hbm.at[wid], idx0.at[pl.ds(0, NCH)])
    pltpu.sync_copy(i1_hbm.at[wid], idx1.at[pl.ds(0, NCH)])
    zi = jnp.zeros((16,), jnp.int32)
    for k in range(CHUNK // 16):
        idx0[NCH, pl.ds(k * 16, 16)] = zi
        idx1[NCH, pl.ds(k * 16, 16)] = zi

    col = lax.iota(jnp.int32, 16) * 16

    def compute(j, ra, rb):
        def group(g, _):
            for e16 in range(16):
                e = g * 16 + e16
                c = ra[e, pl.ds(0, 16)] * rb[e, pl.ds(0, 16)]
                for k in range(1, D // 16):
                    c += ra[e, pl.ds(k * 16, 16)] * rb[e, pl.ds(k * 16, 16)]
                fbuf[pl.ds(e16 * 16, 16)] = c
            acc = plsc.load_gather(fbuf, [col])
            for jj in range(1, 16):
                acc = acc + plsc.load_gather(fbuf, [col + jj])
            lbuf[pl.ds(j * CHUNK + g * 16, 16)] = acc
            return 0

        lax.fori_loop(0, CHUNK // 16, group, 0)

    pltpu.async_copy(z_hbm.at[idx0.at[0]], a0, sa0)
    pltpu.async_copy(z_hbm.at[idx1.at[0]], b0, sb0)

    def pair(i, _):
        j = 2 * i
        pltpu.async_copy(z_hbm.at[idx0.at[j + 1]], a1, sa1)
        pltpu.async_copy(z_hbm.at[idx1.at[j + 1]], b1, sb1)
        pltpu.make_async_copy(z_hbm.at[idx0.at[j]], a0, sa0).wait()
        pltpu.make_async_copy(z_hbm.at[idx1.at[j]], b0, sb0).wait()
        compute(j, a0, b0)
        pltpu.async_copy(z_hbm.at[idx0.at[j + 2]], a0, sa0)
        pltpu.async_copy(z_hbm.at[idx1.at[j + 2]], b0, sb0)
        pltpu.make_async_copy(z_hbm.at[idx0.at[j + 1]], a1, sa1).wait()
        pltpu.make_async_copy(z_hbm.at[idx1.at[j + 1]], b1, sb1).wait()
        compute(j + 1, a1, b1)
        return 0

    lax.fori_loop(0, NCH // 2, pair, 0)
    pltpu.make_async_copy(z_hbm.at[idx0.at[NCH]], a0, sa0).wait()
    pltpu.make_async_copy(z_hbm.at[idx1.at[NCH]], b0, sb0).wait()
    pltpu.sync_copy(lbuf, out_hbm.at[pl.ds(wid * WPE, WPE)])


_logits_call = pl.kernel(
    _logits_body,
    out_type=jax.ShapeDtypeStruct((NW * WPE,), jnp.float32),
    mesh=_mesh,
    compiler_params=_sc_params,
    scratch_types=[
        pltpu.VMEM((NCH + 1, CHUNK), jnp.int32),
        pltpu.VMEM((NCH + 1, CHUNK), jnp.int32),
        pltpu.VMEM((CHUNK, D), jnp.float32),
        pltpu.VMEM((CHUNK, D), jnp.float32),
        pltpu.VMEM((CHUNK, D), jnp.float32),
        pltpu.VMEM((CHUNK, D), jnp.float32),
        pltpu.VMEM((WPE,), jnp.float32),
        pltpu.VMEM((256,), jnp.float32),
        pltpu.SemaphoreType.DMA,
        pltpu.SemaphoreType.DMA,
        pltpu.SemaphoreType.DMA,
        pltpu.SemaphoreType.DMA,
    ],
)


def _tc_prep_body(x_ref, w1_ref, w2_ref, w3_ref, b1_ref, b2_ref,
                  xc_ref, bc1_ref, bc2_ref):
    w23 = jnp.dot(w2_ref[...], w3_ref[...], preferred_element_type=jnp.float32)
    wc = jnp.dot(w1_ref[...], w23, preferred_element_type=jnp.float32)
    xc_ref[...] = jnp.dot(x_ref[...], wc, preferred_element_type=jnp.float32)
    bc1_ref[...] = jnp.dot(b1_ref[...], w23, preferred_element_type=jnp.float32)
    bc2_ref[...] = jnp.dot(b2_ref[...], w3_ref[...],
                           preferred_element_type=jnp.float32)


def _tc_prep(x_pad, W1, W2, W3, b1, b2):
    return pl.pallas_call(
        _tc_prep_body,
        out_shape=[
            jax.ShapeDtypeStruct((NPAD, D), jnp.float32),
            jax.ShapeDtypeStruct((1, D), jnp.float32),
            jax.ShapeDtypeStruct((1, D), jnp.float32),
        ],
    )(x_pad, W1, W2, W3, b1, b2)


def _pad_plan(idx, mod):
    body = idx.reshape(NW, EPW)
    npad = WPE - EPW
    base = N if mod == NSCR else 0
    padv = (base + (jnp.arange(NW * npad, dtype=jnp.int32) % mod)
            ).reshape(NW, npad)
    return jnp.concatenate([body, padv], axis=1).reshape(NW, NCH, CHUNK)


def kernel(x, pos_edge_index, neg_edge_index, W1, b1, W2, b2, W3, b3):
    pos = pos_edge_index.astype(jnp.int32)
    neg = neg_edge_index.astype(jnp.int32)
    src = _pad_plan(jnp.concatenate([pos[0], pos[1]]), NSCR)
    dst = _pad_plan(jnp.concatenate([pos[1], pos[0]]), NSCR)
    sel0 = _pad_plan(jnp.concatenate([pos[0], neg[0]]), N)
    sel1 = _pad_plan(jnp.concatenate([pos[1], neg[1]]), N)

    cst = jnp.zeros((2 * CHUNK, DG), jnp.float32).at[:CHUNK, 0].set(1.0)
    dp = _deg_round(dst, cst)
    deg = dp[0, :, 0] + dp[1, :, 0] + 1.0
    real = jnp.arange(NPAD) < N
    dinv = jnp.where(real, lax.rsqrt(deg), 0.0)
    dinv2 = dinv * dinv

    x_pad = jnp.pad(x, ((0, NPAD - N), (0, 0)))
    xc, bc1, bc2 = _tc_prep(x_pad, W1, W2, W3, b1[None, :], b2[None, :])

    aug = jnp.concatenate(
        [xc, jnp.ones((NPAD, 1), jnp.float32),
         jnp.zeros((NPAD, AUG - D - 1), jnp.float32)], axis=1)
    zcst = jnp.zeros((CHUNK, AUG), jnp.float32)
    s = dinv[:, None] * aug
    p = _round(s, src, dst, zcst)
    tot = p[0] + p[1] + s
    u1 = dinv * tot[:, D]
    s = dinv2[:, None] * tot
    p = _round(s, src, dst, zcst)
    tot = p[0] + p[1] + s
    u2 = dinv * tot[:, D]
    s = dinv2[:, None] * tot
    p = _round(s, src, dst, zcst)
    tot = p[0] + p[1] + s

    z = dinv[:, None] * tot[:, :D] + u2[:, None] * bc1[0] \
        + u1[:, None] * bc2[0] + b3[None, :]

    lp = _logits_call(z[:N], sel0, sel1)
    return lp.reshape(NW, WPE)[:, :EPW].reshape(-1)


# speedup vs baseline: 1.2217x; 1.2217x over previous
import jax
import jax.numpy as jnp
from jax import lax
from jax.experimental import pallas as pl
from jax.experimental.pallas import tpu as pltpu
from jax.experimental.pallas import tpu_sc as plsc

N = 10000
NPAD = 10112
NSCR = NPAD - N
D = 64
AUG = 72
DG = 4
NC = 2
NS = 16
NW = NC * NS
E = 320000
E2 = 2 * E
EPW = E2 // NW
CH = 256
NCH = 79
WPE = NCH * CH
CHD = 128
NCHD = 158
RPT = NPAD // NS
RPTL = N // NS

_mesh = plsc.VectorSubcoreMesh(core_axis_name="c", subcore_axis_name="s")
_sc_params = pltpu.CompilerParams(use_tc_tiling_on_sc=False,
                                  needs_layout_passes=False)


def _round_body(s_hbm, src_hbm, dst_hbm, zcst_hbm, out_hbm,
                idx_s, idx_d, rowbuf, acc):
    cid = lax.axis_index("c")
    sid = lax.axis_index("s")
    wid = sid * NC + cid
    r0 = sid * RPT

    for k in range(RPT // CH):
        pltpu.sync_copy(zcst_hbm, acc.at[pl.ds(r0 + k * CH, CH)])
    pltpu.sync_copy(zcst_hbm.at[pl.ds(0, RPT % CH)],
                    acc.at[pl.ds(r0 + (RPT // CH) * CH, RPT % CH)])
    pltpu.sync_copy(src_hbm.at[wid], idx_s)
    pltpu.sync_copy(dst_hbm.at[wid], idx_d)
    plsc.subcore_barrier()

    def chunk(j, _):
        pltpu.sync_copy(s_hbm.at[idx_s.at[j]], rowbuf)
        pltpu.sync_copy(rowbuf, acc.at[idx_d.at[j]], add=True)
        return 0

    lax.fori_loop(0, NCH, chunk, 0)
    plsc.subcore_barrier()
    pltpu.sync_copy(acc.at[pl.ds(r0, RPT)], out_hbm.at[cid, pl.ds(r0, RPT)])


_round = pl.kernel(
    _round_body,
    out_type=jax.ShapeDtypeStruct((NC, NPAD, AUG), jnp.float32),
    mesh=_mesh,
    compiler_params=_sc_params,
    scratch_types=[
        pltpu.VMEM((NCH, CH), jnp.int32),
        pltpu.VMEM((NCH, CH), jnp.int32),
        pltpu.VMEM((CH, AUG), jnp.float32),
        pltpu.VMEM_SHARED((NPAD, AUG), jnp.float32),
    ],
)


def _deg_body(dst_hbm, cst_hbm, out_hbm, idx_d, ubuf, acc, ss0, ss1):
    cid = lax.axis_index("c")
    sid = lax.axis_index("s")
    wid = sid * NC + cid
    r0 = sid * RPT

    pltpu.sync_copy(cst_hbm, ubuf)
    for k in range(RPT // CHD):
        pltpu.sync_copy(ubuf.at[pl.ds(CHD, CHD)],
                        acc.at[pl.ds(r0 + k * CHD, CHD)])
    if RPT % CHD:
        pltpu.sync_copy(ubuf.at[pl.ds(CHD, RPT % CHD)],
                        acc.at[pl.ds(r0 + (RPT // CHD) * CHD, RPT % CHD)])
    pltpu.sync_copy(dst_hbm.at[wid], idx_d)
    plsc.subcore_barrier()

    def pair(i, _):
        j = 2 * i
        d0 = pltpu.async_copy(ubuf.at[pl.ds(0, CHD)],
                              acc.at[idx_d.at[j]], ss0, add=True)
        d1 = pltpu.async_copy(ubuf.at[pl.ds(0, CHD)],
                              acc.at[idx_d.at[j + 1]], ss1, add=True)
        d0.wait()
        d1.wait()
        return 0

    lax.fori_loop(0, NCHD // 2, pair, 0)
    plsc.subcore_barrier()
    pltpu.sync_copy(acc.at[pl.ds(r0, RPT)], out_hbm.at[cid, pl.ds(r0, RPT)])


_deg_round = pl.kernel(
    _deg_body,
    out_type=jax.ShapeDtypeStruct((NC, NPAD, DG), jnp.float32),
    mesh=_mesh,
    compiler_params=_sc_params,
    scratch_types=[
        pltpu.VMEM((NCHD, CHD), jnp.int32),
        pltpu.VMEM((2 * CHD, DG), jnp.float32),
        pltpu.VMEM_SHARED((NPAD, DG), jnp.float32),
        pltpu.SemaphoreType.DMA,
        pltpu.SemaphoreType.DMA,
    ],
)


def _logits_body(z_hbm, i0_hbm, i1_hbm, out_hbm,
                 idx0, idx1, rowa, rowb, lbuf, fbuf, table):
    cid = lax.axis_index("c")
    sid = lax.axis_index("s")
    wid = sid * NC + cid
    r0 = sid * RPTL

    pltpu.sync_copy(z_hbm.at[pl.ds(r0, RPTL)], table.at[pl.ds(r0, RPTL)])
    pltpu.sync_copy(i0_hbm.at[wid], idx0)
    pltpu.sync_copy(i1_hbm.at[wid], idx1)
    plsc.subcore_barrier()

    col = lax.iota(jnp.int32, 16) * 16

    def chunk(j, _):
        pltpu.sync_copy(table.at[idx0.at[j]], rowa)
        pltpu.sync_copy(table.at[idx1.at[j]], rowb)

        def group(g, _):
            for e16 in range(16):
                e = g * 16 + e16
                c = rowa[e, pl.ds(0, 16)] * rowb[e, pl.ds(0, 16)]
                for k in range(1, D // 16):
                    c += rowa[e, pl.ds(k * 16, 16)] * rowb[e, pl.ds(k * 16, 16)]
                fbuf[pl.ds(e16 * 16, 16)] = c
            acc = plsc.load_gather(fbuf, [col])
            for jj in range(1, 16):
                acc = acc + plsc.load_gather(fbuf, [col + jj])
            lbuf[pl.ds(g * 16, 16)] = acc
            return 0

        lax.fori_loop(0, CH // 16, group, 0)
        pltpu.sync_copy(lbuf, out_hbm.at[pl.ds(wid * WPE + j * CH, CH)])
        return 0

    lax.fori_loop(0, NCH, chunk, 0)


_logits_call = pl.kernel(
    _logits_body,
    out_type=jax.ShapeDtypeStruct((NW * WPE,), jnp.float32),
    mesh=_mesh,
    compiler_params=_sc_params,
    scratch_types=[
        pltpu.VMEM((NCH, CH), jnp.int32),
        pltpu.VMEM((NCH, CH), jnp.int32),
        pltpu.VMEM((CH, D), jnp.float32),
        pltpu.VMEM((CH, D), jnp.float32),
        pltpu.VMEM((CH,), jnp.float32),
        pltpu.VMEM((256,), jnp.float32),
        pltpu.VMEM_SHARED((N, D), jnp.float32),
    ],
)


def _tc_prep_body(x_ref, w1_ref, w2_ref, w3_ref, b1_ref, b2_ref,
                  xc_ref, bc1_ref, bc2_ref):
    w23 = jnp.dot(w2_ref[...], w3_ref[...], preferred_element_type=jnp.float32)
    wc = jnp.dot(w1_ref[...], w23, preferred_element_type=jnp.float32)
    xc_ref[...] = jnp.dot(x_ref[...], wc, preferred_element_type=jnp.float32)
    bc1_ref[...] = jnp.dot(b1_ref[...], w23, preferred_element_type=jnp.float32)
    bc2_ref[...] = jnp.dot(b2_ref[...], w3_ref[...],
                           preferred_element_type=jnp.float32)


def _tc_prep(x_pad, W1, W2, W3, b1, b2):
    return pl.pallas_call(
        _tc_prep_body,
        out_shape=[
            jax.ShapeDtypeStruct((NPAD, D), jnp.float32),
            jax.ShapeDtypeStruct((1, D), jnp.float32),
            jax.ShapeDtypeStruct((1, D), jnp.float32),
        ],
    )(x_pad, W1, W2, W3, b1, b2)


def _pad_plan(idx, mod, ch):
    body = idx.reshape(NW, EPW)
    npad = WPE - EPW
    base = N if mod == NSCR else 0
    padv = (base + (jnp.arange(NW * npad, dtype=jnp.int32) % mod)
            ).reshape(NW, npad)
    return jnp.concatenate([body, padv], axis=1).reshape(NW, WPE // ch, ch)


def kernel(x, pos_edge_index, neg_edge_index, W1, b1, W2, b2, W3, b3):
    pos = pos_edge_index.astype(jnp.int32)
    neg = neg_edge_index.astype(jnp.int32)
    src = _pad_plan(jnp.concatenate([pos[0], pos[1]]), NSCR, CH)
    dst = _pad_plan(jnp.concatenate([pos[1], pos[0]]), NSCR, CH)
    dstd = _pad_plan(jnp.concatenate([pos[1], pos[0]]), NSCR, CHD)
    sel0 = _pad_plan(jnp.concatenate([pos[0], neg[0]]), N, CH)
    sel1 = _pad_plan(jnp.concatenate([pos[1], neg[1]]), N, CH)

    cst = jnp.zeros((2 * CHD, DG), jnp.float32).at[:CHD, 0].set(1.0)
    dp = _deg_round(dstd, cst)
    deg = dp[0, :, 0] + dp[1, :, 0] + 1.0
    real = jnp.arange(NPAD) < N
    dinv = jnp.where(real, lax.rsqrt(deg), 0.0)
    dinv2 = dinv * dinv

    x_pad = jnp.pad(x, ((0, NPAD - N), (0, 0)))
    xc, bc1, bc2 = _tc_prep(x_pad, W1, W2, W3, b1[None, :], b2[None, :])

    aug = jnp.concatenate(
        [xc, jnp.ones((NPAD, 1), jnp.float32),
         jnp.zeros((NPAD, AUG - D - 1), jnp.float32)], axis=1)
    zcst = jnp.zeros((CH, AUG), jnp.float32)
    s = dinv[:, None] * aug
    p = _round(s, src, dst, zcst)
    tot = p[0] + p[1] + s
    u1 = dinv * tot[:, D]
    s = dinv2[:, None] * tot
    p = _round(s, src, dst, zcst)
    tot = p[0] + p[1] + s
    u2 = dinv * tot[:, D]
    s = dinv2[:, None] * tot
    p = _round(s, src, dst, zcst)
    tot = p[0] + p[1] + s

    z = dinv[:, None] * tot[:, :D] + u2[:, None] * bc1[0] \
        + u1[:, None] * bc2[0] + b3[None, :]

    lp = _logits_call(z[:N], sel0, sel1)
    return lp.reshape(NW, WPE)[:, :EPW].reshape(-1)


# speedup vs baseline: 1.2848x; 1.0516x over previous
import jax
import jax.numpy as jnp
from jax import lax
from jax.experimental import pallas as pl
from jax.experimental.pallas import tpu as pltpu
from jax.experimental.pallas import tpu_sc as plsc

N = 10000
NPAD = 10112
NSCR = NPAD - N
D = 64
AUG = 72
DG = 4
NC = 2
NS = 16
NW = NC * NS
E = 320000
E2 = 2 * E
EPW = E2 // NW
CH = 128
NCH = 160
WPE = NCH * CH
RPT = NPAD // NS
RPTL = N // NS

_mesh = plsc.VectorSubcoreMesh(core_axis_name="c", subcore_axis_name="s")
_sc_params = pltpu.CompilerParams(use_tc_tiling_on_sc=False,
                                  needs_layout_passes=False)


def _round_body(s_hbm, src_hbm, dst_hbm, zcst_hbm, out_hbm,
                idx_s, idx_d, b0, b1, b2, b3_, acc,
                sg0, sg1, sg2, sg3, ss0, ss1, ss2, ss3):
    cid = lax.axis_index("c")
    sid = lax.axis_index("s")
    wid = sid * NC + cid
    r0 = sid * RPT
    bufs = (b0, b1, b2, b3_)
    sgs = (sg0, sg1, sg2, sg3)
    sss = (ss0, ss1, ss2, ss3)

    for k in range(RPT // CH):
        pltpu.sync_copy(zcst_hbm, acc.at[pl.ds(r0 + k * CH, CH)])
    if RPT % CH:
        pltpu.sync_copy(zcst_hbm.at[pl.ds(0, RPT % CH)],
                        acc.at[pl.ds(r0 + (RPT // CH) * CH, RPT % CH)])
    pltpu.sync_copy(src_hbm.at[wid], idx_s)
    pltpu.sync_copy(dst_hbm.at[wid], idx_d)
    plsc.subcore_barrier()

    def quad(i, _):
        j = 4 * i
        gs = [pltpu.async_copy(s_hbm.at[idx_s.at[j + k]], bufs[k], sgs[k])
              for k in range(4)]
        ss = []
        for k in range(4):
            gs[k].wait()
            ss.append(pltpu.async_copy(bufs[k], acc.at[idx_d.at[j + k]],
                                       sss[k], add=True))
        for k in range(4):
            ss[k].wait()
        return 0

    lax.fori_loop(0, NCH // 4, quad, 0)
    plsc.subcore_barrier()
    pltpu.sync_copy(acc.at[pl.ds(r0, RPT)], out_hbm.at[cid, pl.ds(r0, RPT)])


_round = pl.kernel(
    _round_body,
    out_type=jax.ShapeDtypeStruct((NC, NPAD, AUG), jnp.float32),
    mesh=_mesh,
    compiler_params=_sc_params,
    scratch_types=(
        [pltpu.VMEM((NCH, CH), jnp.int32)] * 2
        + [pltpu.VMEM((CH, AUG), jnp.float32)] * 4
        + [pltpu.VMEM_SHARED((NPAD, AUG), jnp.float32)]
        + [pltpu.SemaphoreType.DMA] * 8
    ),
)


def _deg_body(dst_hbm, cst_hbm, out_hbm, idx_d, ubuf, acc,
              ss0, ss1, ss2, ss3):
    cid = lax.axis_index("c")
    sid = lax.axis_index("s")
    wid = sid * NC + cid
    r0 = sid * RPT
    sss = (ss0, ss1, ss2, ss3)

    pltpu.sync_copy(cst_hbm, ubuf)
    for k in range(RPT // CH):
        pltpu.sync_copy(ubuf.at[pl.ds(CH, CH)],
                        acc.at[pl.ds(r0 + k * CH, CH)])
    if RPT % CH:
        pltpu.sync_copy(ubuf.at[pl.ds(CH, RPT % CH)],
                        acc.at[pl.ds(r0 + (RPT // CH) * CH, RPT % CH)])
    pltpu.sync_copy(dst_hbm.at[wid], idx_d)
    plsc.subcore_barrier()

    def quad(i, _):
        j = 4 * i
        ds_ = [pltpu.async_copy(ubuf.at[pl.ds(0, CH)],
                                acc.at[idx_d.at[j + k]], sss[k], add=True)
               for k in range(4)]
        for d in ds_:
            d.wait()
        return 0

    lax.fori_loop(0, NCH // 4, quad, 0)
    plsc.subcore_barrier()
    pltpu.sync_copy(acc.at[pl.ds(r0, RPT)], out_hbm.at[cid, pl.ds(r0, RPT)])


_deg_round = pl.kernel(
    _deg_body,
    out_type=jax.ShapeDtypeStruct((NC, NPAD, DG), jnp.float32),
    mesh=_mesh,
    compiler_params=_sc_params,
    scratch_types=(
        [pltpu.VMEM((NCH, CH), jnp.int32),
         pltpu.VMEM((2 * CH, DG), jnp.float32),
         pltpu.VMEM_SHARED((NPAD, DG), jnp.float32)]
        + [pltpu.SemaphoreType.DMA] * 4
    ),
)


def _logits_body(z_hbm, i0_hbm, i1_hbm, out_hbm,
                 idx0, idx1, a0, a1, a2, a3, b0, b1, b2, b3_, lbuf, fbuf,
                 sa0, sa1, sa2, sa3, sb0, sb1, sb2, sb3):
    cid = lax.axis_index("c")
    sid = lax.axis_index("s")
    wid = sid * NC + cid
    abufs = (a0, a1, a2, a3)
    bbufs = (b0, b1, b2, b3_)
    sas = (sa0, sa1, sa2, sa3)
    sbs = (sb0, sb1, sb2, sb3)

    pltpu.sync_copy(i0_hbm.at[wid], idx0)
    pltpu.sync_copy(i1_hbm.at[wid], idx1)

    col = lax.iota(jnp.int32, 16) * 16

    def compute(ra, rb):
        def group(g, _):
            for e16 in range(16):
                e = g * 16 + e16
                c = ra[e, pl.ds(0, 16)] * rb[e, pl.ds(0, 16)]
                for k in range(1, D // 16):
                    c += ra[e, pl.ds(k * 16, 16)] * rb[e, pl.ds(k * 16, 16)]
                fbuf[pl.ds(e16 * 16, 16)] = c
            acc = plsc.load_gather(fbuf, [col])
            for jj in range(1, 16):
                acc = acc + plsc.load_gather(fbuf, [col + jj])
            lbuf[pl.ds(g * 16, 16)] = acc
            return 0

        lax.fori_loop(0, CH // 16, group, 0)

    def quad(i, _):
        j = 4 * i
        gas = [pltpu.async_copy(z_hbm.at[idx0.at[j + k]], abufs[k], sas[k])
               for k in range(4)]
        gbs = [pltpu.async_copy(z_hbm.at[idx1.at[j + k]], bbufs[k], sbs[k])
               for k in range(4)]
        for k in range(4):
            gas[k].wait()
            gbs[k].wait()
            compute(abufs[k], bbufs[k])
            pltpu.sync_copy(lbuf,
                            out_hbm.at[pl.ds(wid * WPE + (j + k) * CH, CH)])
        return 0

    lax.fori_loop(0, NCH // 4, quad, 0)


_logits_call = pl.kernel(
    _logits_body,
    out_type=jax.ShapeDtypeStruct((NW * WPE,), jnp.float32),
    mesh=_mesh,
    compiler_params=_sc_params,
    scratch_types=(
        [pltpu.VMEM((NCH, CH), jnp.int32)] * 2
        + [pltpu.VMEM((CH, D), jnp.float32)] * 8
        + [pltpu.VMEM((CH,), jnp.float32),
           pltpu.VMEM((256,), jnp.float32)]
        + [pltpu.SemaphoreType.DMA] * 8
    ),
)


def _tc_prep_body(x_ref, w1_ref, w2_ref, w3_ref, b1_ref, b2_ref,
                  xc_ref, bc1_ref, bc2_ref):
    w23 = jnp.dot(w2_ref[...], w3_ref[...], preferred_element_type=jnp.float32)
    wc = jnp.dot(w1_ref[...], w23, preferred_element_type=jnp.float32)
    xc_ref[...] = jnp.dot(x_ref[...], wc, preferred_element_type=jnp.float32)
    bc1_ref[...] = jnp.dot(b1_ref[...], w23, preferred_element_type=jnp.float32)
    bc2_ref[...] = jnp.dot(b2_ref[...], w3_ref[...],
                           preferred_element_type=jnp.float32)


def _tc_prep(x_pad, W1, W2, W3, b1, b2):
    return pl.pallas_call(
        _tc_prep_body,
        out_shape=[
            jax.ShapeDtypeStruct((NPAD, D), jnp.float32),
            jax.ShapeDtypeStruct((1, D), jnp.float32),
            jax.ShapeDtypeStruct((1, D), jnp.float32),
        ],
    )(x_pad, W1, W2, W3, b1, b2)


def _pad_plan(idx, mod):
    body = idx.reshape(NW, EPW)
    npad = WPE - EPW
    base = N if mod == NSCR else 0
    padv = (base + (jnp.arange(NW * npad, dtype=jnp.int32) % mod)
            ).reshape(NW, npad)
    return jnp.concatenate([body, padv], axis=1).reshape(NW, NCH, CH)


def kernel(x, pos_edge_index, neg_edge_index, W1, b1, W2, b2, W3, b3):
    pos = pos_edge_index.astype(jnp.int32)
    neg = neg_edge_index.astype(jnp.int32)
    src = _pad_plan(jnp.concatenate([pos[0], pos[1]]), NSCR)
    dst = _pad_plan(jnp.concatenate([pos[1], pos[0]]), NSCR)
    sel0 = _pad_plan(jnp.concatenate([pos[0], neg[0]]), N)
    sel1 = _pad_plan(jnp.concatenate([pos[1], neg[1]]), N)

    cst = jnp.zeros((2 * CH, DG), jnp.float32).at[:CH, 0].set(1.0)
    dp = _deg_round(dst, cst)
    deg = dp[0, :, 0] + dp[1, :, 0] + 1.0
    real = jnp.arange(NPAD) < N
    dinv = jnp.where(real, lax.rsqrt(deg), 0.0)
    dinv2 = dinv * dinv

    x_pad = jnp.pad(x, ((0, NPAD - N), (0, 0)))
    xc, bc1, bc2 = _tc_prep(x_pad, W1, W2, W3, b1[None, :], b2[None, :])

    aug = jnp.concatenate(
        [xc, jnp.ones((NPAD, 1), jnp.float32),
         jnp.zeros((NPAD, AUG - D - 1), jnp.float32)], axis=1)
    zcst = jnp.zeros((CH, AUG), jnp.float32)
    s = dinv[:, None] * aug
    p = _round(s, src, dst, zcst)
    tot = p[0] + p[1] + s
    u1 = dinv * tot[:, D]
    s = dinv2[:, None] * tot
    p = _round(s, src, dst, zcst)
    tot = p[0] + p[1] + s
    u2 = dinv * tot[:, D]
    s = dinv2[:, None] * tot
    p = _round(s, src, dst, zcst)
    tot = p[0] + p[1] + s

    z = dinv[:, None] * tot[:, :D] + u2[:, None] * bc1[0] \
        + u1[:, None] * bc2[0] + b3[None, :]

    lp = _logits_call(z[:N], sel0, sel1)
    return lp.reshape(NW, WPE)[:, :EPW].reshape(-1)


# speedup vs baseline: 1.2982x; 1.0104x over previous
import jax
import jax.numpy as jnp
from jax import lax
from jax.experimental import pallas as pl
from jax.experimental.pallas import tpu as pltpu
from jax.experimental.pallas import tpu_sc as plsc

N = 10000
NPAD = 10240
NSCR = NPAD - N
D = 64
AUG = 72
DG = 8
NC = 2
NS = 16
NW = NC * NS
E = 320000
E2 = 2 * E
EPW = E2 // NW
CH = 128
NCH = 160
WPE = NCH * CH
RPT = NPAD // NS
RPTL = N // NS

_mesh = plsc.VectorSubcoreMesh(core_axis_name="c", subcore_axis_name="s")
_sc_params = pltpu.CompilerParams(use_tc_tiling_on_sc=False,
                                  needs_layout_passes=False)


def _round_body(s_hbm, src_hbm, dst_hbm, zcst_hbm, out_hbm,
                idx_s, idx_d, b0, b1, b2, b3_, acc,
                sg0, sg1, sg2, sg3, ss0, ss1, ss2, ss3):
    cid = lax.axis_index("c")
    sid = lax.axis_index("s")
    wid = sid * NC + cid
    r0 = sid * RPT
    bufs = (b0, b1, b2, b3_)
    sgs = (sg0, sg1, sg2, sg3)
    sss = (ss0, ss1, ss2, ss3)

    for k in range(RPT // CH):
        pltpu.sync_copy(zcst_hbm, acc.at[pl.ds(r0 + k * CH, CH)])
    if RPT % CH:
        pltpu.sync_copy(zcst_hbm.at[pl.ds(0, RPT % CH)],
                        acc.at[pl.ds(r0 + (RPT // CH) * CH, RPT % CH)])
    pltpu.sync_copy(src_hbm.at[wid], idx_s)
    pltpu.sync_copy(dst_hbm.at[wid], idx_d)
    plsc.subcore_barrier()

    def quad(i, _):
        j = 4 * i
        gs = [pltpu.async_copy(s_hbm.at[idx_s.at[j + k]], bufs[k], sgs[k])
              for k in range(4)]
        ss = []
        for k in range(4):
            gs[k].wait()
            ss.append(pltpu.async_copy(bufs[k], acc.at[idx_d.at[j + k]],
                                       sss[k], add=True))
        for k in range(4):
            ss[k].wait()
        return 0

    lax.fori_loop(0, NCH // 4, quad, 0)
    plsc.subcore_barrier()
    pltpu.sync_copy(acc.at[pl.ds(r0, RPT)], out_hbm.at[cid, pl.ds(r0, RPT)])


_round = pl.kernel(
    _round_body,
    out_type=jax.ShapeDtypeStruct((NC, NPAD, AUG), jnp.float32),
    mesh=_mesh,
    compiler_params=_sc_params,
    scratch_types=(
        [pltpu.VMEM((NCH, CH), jnp.int32)] * 2
        + [pltpu.VMEM((CH, AUG), jnp.float32)] * 4
        + [pltpu.VMEM_SHARED((NPAD, AUG), jnp.float32)]
        + [pltpu.SemaphoreType.DMA] * 8
    ),
)


def _deg_body(dst_hbm, cst_hbm, out_hbm, idx_d, ubuf, acc,
              ss0, ss1, ss2, ss3):
    cid = lax.axis_index("c")
    sid = lax.axis_index("s")
    wid = sid * NC + cid
    r0 = sid * RPT
    sss = (ss0, ss1, ss2, ss3)

    pltpu.sync_copy(cst_hbm, ubuf)
    for k in range(RPT // CH):
        pltpu.sync_copy(ubuf.at[pl.ds(CH, CH)],
                        acc.at[pl.ds(r0 + k * CH, CH)])
    if RPT % CH:
        pltpu.sync_copy(ubuf.at[pl.ds(CH, RPT % CH)],
                        acc.at[pl.ds(r0 + (RPT // CH) * CH, RPT % CH)])
    pltpu.sync_copy(dst_hbm.at[wid], idx_d)
    plsc.subcore_barrier()

    def quad(i, _):
        j = 4 * i
        ds_ = [pltpu.async_copy(ubuf.at[pl.ds(0, CH)],
                                acc.at[idx_d.at[j + k]], sss[k], add=True)
               for k in range(4)]
        for d in ds_:
            d.wait()
        return 0

    lax.fori_loop(0, NCH // 4, quad, 0)
    plsc.subcore_barrier()
    pltpu.sync_copy(acc.at[pl.ds(r0, RPT)], out_hbm.at[cid, pl.ds(r0, RPT)])


_deg_round = pl.kernel(
    _deg_body,
    out_type=jax.ShapeDtypeStruct((NC, NPAD, DG), jnp.float32),
    mesh=_mesh,
    compiler_params=_sc_params,
    scratch_types=(
        [pltpu.VMEM((NCH, CH), jnp.int32),
         pltpu.VMEM((2 * CH, DG), jnp.float32),
         pltpu.VMEM_SHARED((NPAD, DG), jnp.float32)]
        + [pltpu.SemaphoreType.DMA] * 4
    ),
)


def _logits_body(z_hbm, i0_hbm, i1_hbm, out_hbm,
                 idx0, idx1, a0, a1, a2, a3, b0, b1, b2, b3_, lbuf, fbuf,
                 sa0, sa1, sa2, sa3, sb0, sb1, sb2, sb3):
    cid = lax.axis_index("c")
    sid = lax.axis_index("s")
    wid = sid * NC + cid
    abufs = (a0, a1, a2, a3)
    bbufs = (b0, b1, b2, b3_)
    sas = (sa0, sa1, sa2, sa3)
    sbs = (sb0, sb1, sb2, sb3)

    pltpu.sync_copy(i0_hbm.at[wid], idx0)
    pltpu.sync_copy(i1_hbm.at[wid], idx1)

    col = lax.iota(jnp.int32, 16) * 16

    def compute(ra, rb):
        def group(g, _):
            for e16 in range(16):
                e = g * 16 + e16
                c = ra[e, pl.ds(0, 16)] * rb[e, pl.ds(0, 16)]
                for k in range(1, D // 16):
                    c += ra[e, pl.ds(k * 16, 16)] * rb[e, pl.ds(k * 16, 16)]
                fbuf[pl.ds(e16 * 16, 16)] = c
            acc = plsc.load_gather(fbuf, [col])
            for jj in range(1, 16):
                acc = acc + plsc.load_gather(fbuf, [col + jj])
            lbuf[pl.ds(g * 16, 16)] = acc
            return 0

        lax.fori_loop(0, CH // 16, group, 0)

    def quad(i, _):
        j = 4 * i
        gas = [pltpu.async_copy(z_hbm.at[idx0.at[j + k]], abufs[k], sas[k])
               for k in range(4)]
        gbs = [pltpu.async_copy(z_hbm.at[idx1.at[j + k]], bbufs[k], sbs[k])
               for k in range(4)]
        for k in range(4):
            gas[k].wait()
            gbs[k].wait()
            compute(abufs[k], bbufs[k])
            pltpu.sync_copy(lbuf,
                            out_hbm.at[pl.ds(wid * WPE + (j + k) * CH, CH)])
        return 0

    lax.fori_loop(0, NCH // 4, quad, 0)


_logits_call = pl.kernel(
    _logits_body,
    out_type=jax.ShapeDtypeStruct((NW * WPE,), jnp.float32),
    mesh=_mesh,
    compiler_params=_sc_params,
    scratch_types=(
        [pltpu.VMEM((NCH, CH), jnp.int32)] * 2
        + [pltpu.VMEM((CH, D), jnp.float32)] * 8
        + [pltpu.VMEM((CH,), jnp.float32),
           pltpu.VMEM((256,), jnp.float32)]
        + [pltpu.SemaphoreType.DMA] * 8
    ),
)


def _tc_prep_body(x_ref, w1_ref, w2_ref, w3_ref, b1_ref, b2_ref,
                  xc_ref, bc1_ref, bc2_ref):
    w23 = jnp.dot(w2_ref[...], w3_ref[...], preferred_element_type=jnp.float32)
    wc = jnp.dot(w1_ref[...], w23, preferred_element_type=jnp.float32)
    xc_ref[...] = jnp.dot(x_ref[...], wc, preferred_element_type=jnp.float32)
    bc1_ref[...] = jnp.dot(b1_ref[...], w23, preferred_element_type=jnp.float32)
    bc2_ref[...] = jnp.dot(b2_ref[...], w3_ref[...],
                           preferred_element_type=jnp.float32)


def _tc_prep(x_pad, W1, W2, W3, b1, b2):
    return pl.pallas_call(
        _tc_prep_body,
        out_shape=[
            jax.ShapeDtypeStruct((NPAD, D), jnp.float32),
            jax.ShapeDtypeStruct((1, D), jnp.float32),
            jax.ShapeDtypeStruct((1, D), jnp.float32),
        ],
    )(x_pad, W1, W2, W3, b1, b2)


def _pad_plan(idx, mod):
    body = idx.reshape(NW, EPW)
    npad = WPE - EPW
    base = N if mod == NSCR else 0
    padv = (base + (jnp.arange(NW * npad, dtype=jnp.int32) % mod)
            ).reshape(NW, npad)
    return jnp.concatenate([body, padv], axis=1).reshape(NW, NCH, CH)


def kernel(x, pos_edge_index, neg_edge_index, W1, b1, W2, b2, W3, b3):
    pos = pos_edge_index.astype(jnp.int32)
    neg = neg_edge_index.astype(jnp.int32)
    src = _pad_plan(jnp.concatenate([pos[0], pos[1]]), NSCR)
    dst = _pad_plan(jnp.concatenate([pos[1], pos[0]]), NSCR)
    sel0 = _pad_plan(jnp.concatenate([pos[0], neg[0]]), N)
    sel1 = _pad_plan(jnp.concatenate([pos[1], neg[1]]), N)

    cst = jnp.zeros((2 * CH, DG), jnp.float32).at[:CH, 0].set(1.0)
    dp = _deg_round(dst, cst)
    deg = dp[0, :, 0] + dp[1, :, 0] + 1.0
    real = jnp.arange(NPAD) < N
    dinv = jnp.where(real, lax.rsqrt(deg), 0.0)
    dinv2 = dinv * dinv

    x_pad = jnp.pad(x, ((0, NPAD - N), (0, 0)))
    xc, bc1, bc2 = _tc_prep(x_pad, W1, W2, W3, b1[None, :], b2[None, :])

    aug = jnp.concatenate(
        [xc, jnp.ones((NPAD, 1), jnp.float32),
         jnp.zeros((NPAD, AUG - D - 1), jnp.float32)], axis=1)
    zcst = jnp.zeros((CH, AUG), jnp.float32)
    s = dinv[:, None] * aug
    p = _round(s, src, dst, zcst)
    tot = p[0] + p[1] + s
    u1 = dinv * tot[:, D]
    s = dinv2[:, None] * tot
    p = _round(s, src, dst, zcst)
    tot = p[0] + p[1] + s
    u2 = dinv * tot[:, D]
    s = dinv2[:, None] * tot
    p = _round(s, src, dst, zcst)
    tot = p[0] + p[1] + s

    z = dinv[:, None] * tot[:, :D] + u2[:, None] * bc1[0] \
        + u1[:, None] * bc2[0] + b3[None, :]

    lp = _logits_call(z[:N], sel0, sel1)
    return lp.reshape(NW, WPE)[:, :EPW].reshape(-1)


# speedup vs baseline: 1.3187x; 1.0158x over previous
import jax
import jax.numpy as jnp
from jax import lax
from jax.experimental import pallas as pl
from jax.experimental.pallas import tpu as pltpu
from jax.experimental.pallas import tpu_sc as plsc

N = 10000
NPAD = 10240
NSCR = NPAD - N
D = 64
AUG = 72
DG = 8
NC = 2
NS = 16
NW = NC * NS
E = 320000
E2 = 2 * E
EPW = E2 // NW
CH = 128
NCH = 160
WPE = NCH * CH
RPT = NPAD // NS
RPTL = N // NS

_mesh = plsc.VectorSubcoreMesh(core_axis_name="c", subcore_axis_name="s")
_sc_params = pltpu.CompilerParams(use_tc_tiling_on_sc=False,
                                  needs_layout_passes=False)


def _round_body(s_hbm, src_hbm, dst_hbm, zcst_hbm, out_hbm,
                idx_s, idx_d, b0, b1, b2, b3_, acc,
                sg0, sg1, sg2, sg3, ss0, ss1, ss2, ss3):
    cid = lax.axis_index("c")
    sid = lax.axis_index("s")
    wid = sid * NC + cid
    r0 = sid * RPT
    bufs = (b0, b1, b2, b3_)
    sgs = (sg0, sg1, sg2, sg3)
    sss = (ss0, ss1, ss2, ss3)

    for k in range(RPT // CH):
        pltpu.sync_copy(zcst_hbm, acc.at[pl.ds(r0 + k * CH, CH)])
    if RPT % CH:
        pltpu.sync_copy(zcst_hbm.at[pl.ds(0, RPT % CH)],
                        acc.at[pl.ds(r0 + (RPT // CH) * CH, RPT % CH)])
    pltpu.sync_copy(src_hbm.at[wid], idx_s)
    pltpu.sync_copy(dst_hbm.at[wid], idx_d)
    plsc.subcore_barrier()

    def quad(i, _):
        j = 4 * i
        gs = [pltpu.async_copy(s_hbm.at[idx_s.at[j + k]], bufs[k], sgs[k])
              for k in range(4)]
        ss = []
        for k in range(4):
            gs[k].wait()
            ss.append(pltpu.async_copy(bufs[k], acc.at[idx_d.at[j + k]],
                                       sss[k], add=True))
        for k in range(4):
            ss[k].wait()
        return 0

    lax.fori_loop(0, NCH // 4, quad, 0)
    plsc.subcore_barrier()
    pltpu.sync_copy(acc.at[pl.ds(r0, RPT)], out_hbm.at[cid, pl.ds(r0, RPT)])


_round = pl.kernel(
    _round_body,
    out_type=jax.ShapeDtypeStruct((NC, NPAD, AUG), jnp.float32),
    mesh=_mesh,
    compiler_params=_sc_params,
    scratch_types=(
        [pltpu.VMEM((NCH, CH), jnp.int32)] * 2
        + [pltpu.VMEM((CH, AUG), jnp.float32)] * 4
        + [pltpu.VMEM_SHARED((NPAD, AUG), jnp.float32)]
        + [pltpu.SemaphoreType.DMA] * 8
    ),
)


def _deg_body(dst_hbm, cst_hbm, out_hbm, idx_d, ubuf, acc,
              ss0, ss1, ss2, ss3):
    cid = lax.axis_index("c")
    sid = lax.axis_index("s")
    wid = sid * NC + cid
    r0 = sid * RPT
    sss = (ss0, ss1, ss2, ss3)

    pltpu.sync_copy(cst_hbm, ubuf)
    for k in range(RPT // CH):
        pltpu.sync_copy(ubuf.at[pl.ds(CH, CH)],
                        acc.at[pl.ds(r0 + k * CH, CH)])
    if RPT % CH:
        pltpu.sync_copy(ubuf.at[pl.ds(CH, RPT % CH)],
                        acc.at[pl.ds(r0 + (RPT // CH) * CH, RPT % CH)])
    pltpu.sync_copy(dst_hbm.at[wid], idx_d)
    plsc.subcore_barrier()

    def quad(i, _):
        j = 4 * i
        ds_ = [pltpu.async_copy(ubuf.at[pl.ds(0, CH)],
                                acc.at[idx_d.at[j + k]], sss[k], add=True)
               for k in range(4)]
        for d in ds_:
            d.wait()
        return 0

    lax.fori_loop(0, NCH // 4, quad, 0)
    plsc.subcore_barrier()
    pltpu.sync_copy(acc.at[pl.ds(r0, RPT)], out_hbm.at[cid, pl.ds(r0, RPT)])


_deg_round = pl.kernel(
    _deg_body,
    out_type=jax.ShapeDtypeStruct((NC, NPAD, DG), jnp.float32),
    mesh=_mesh,
    compiler_params=_sc_params,
    scratch_types=(
        [pltpu.VMEM((NCH, CH), jnp.int32),
         pltpu.VMEM((2 * CH, DG), jnp.float32),
         pltpu.VMEM_SHARED((NPAD, DG), jnp.float32)]
        + [pltpu.SemaphoreType.DMA] * 4
    ),
)


def _logits_body(z_hbm, i0_hbm, i1_hbm, out_hbm,
                 idx0, idx1, a0, a1, a2, a3, b0, b1, b2, b3_, lbuf, fbuf,
                 sa0, sa1, sa2, sa3, sb0, sb1, sb2, sb3):
    cid = lax.axis_index("c")
    sid = lax.axis_index("s")
    wid = sid * NC + cid
    abufs = (a0, a1, a2, a3)
    bbufs = (b0, b1, b2, b3_)
    sas = (sa0, sa1, sa2, sa3)
    sbs = (sb0, sb1, sb2, sb3)

    pltpu.sync_copy(i0_hbm.at[wid], idx0)
    pltpu.sync_copy(i1_hbm.at[wid], idx1)

    col = lax.iota(jnp.int32, 16) * 17

    def compute(ra, rb):
        def group(g, _):
            for e16 in range(16):
                e = g * 16 + e16
                c = ra[e, pl.ds(0, 16)] * rb[e, pl.ds(0, 16)]
                for k in range(1, D // 16):
                    c += ra[e, pl.ds(k * 16, 16)] * rb[e, pl.ds(k * 16, 16)]
                fbuf[pl.ds(e16 * 17, 16)] = c
            acc = plsc.load_gather(fbuf, [col])
            for jj in range(1, 16):
                acc = acc + plsc.load_gather(fbuf, [col + jj])
            lbuf[pl.ds(g * 16, 16)] = acc
            return 0

        lax.fori_loop(0, CH // 16, group, 0)

    def quad(i, _):
        j = 4 * i
        gas = [pltpu.async_copy(z_hbm.at[idx0.at[j + k]], abufs[k], sas[k])
               for k in range(4)]
        gbs = [pltpu.async_copy(z_hbm.at[idx1.at[j + k]], bbufs[k], sbs[k])
               for k in range(4)]
        for k in range(4):
            gas[k].wait()
            gbs[k].wait()
            compute(abufs[k], bbufs[k])
            pltpu.sync_copy(lbuf,
                            out_hbm.at[pl.ds(wid * WPE + (j + k) * CH, CH)])
        return 0

    lax.fori_loop(0, NCH // 4, quad, 0)


_logits_call = pl.kernel(
    _logits_body,
    out_type=jax.ShapeDtypeStruct((NW * WPE,), jnp.float32),
    mesh=_mesh,
    compiler_params=_sc_params,
    scratch_types=(
        [pltpu.VMEM((NCH, CH), jnp.int32)] * 2
        + [pltpu.VMEM((CH, D), jnp.float32)] * 8
        + [pltpu.VMEM((CH,), jnp.float32),
           pltpu.VMEM((16 * 17, ), jnp.float32)]
        + [pltpu.SemaphoreType.DMA] * 8
    ),
)


def _tc_prep_body(x_ref, w1_ref, w2_ref, w3_ref, b1_ref, b2_ref,
                  xc_ref, bc1_ref, bc2_ref):
    w23 = jnp.dot(w2_ref[...], w3_ref[...], preferred_element_type=jnp.float32)
    wc = jnp.dot(w1_ref[...], w23, preferred_element_type=jnp.float32)
    xc_ref[...] = jnp.dot(x_ref[...], wc, preferred_element_type=jnp.float32)
    bc1_ref[...] = jnp.dot(b1_ref[...], w23, preferred_element_type=jnp.float32)
    bc2_ref[...] = jnp.dot(b2_ref[...], w3_ref[...],
                           preferred_element_type=jnp.float32)


def _tc_prep(x_pad, W1, W2, W3, b1, b2):
    return pl.pallas_call(
        _tc_prep_body,
        out_shape=[
            jax.ShapeDtypeStruct((NPAD, D), jnp.float32),
            jax.ShapeDtypeStruct((1, D), jnp.float32),
            jax.ShapeDtypeStruct((1, D), jnp.float32),
        ],
    )(x_pad, W1, W2, W3, b1, b2)


def _pad_plan(idx, mod):
    body = idx.reshape(NW, EPW)
    npad = WPE - EPW
    base = N if mod == NSCR else 0
    padv = (base + (jnp.arange(NW * npad, dtype=jnp.int32) % mod)
            ).reshape(NW, npad)
    return jnp.concatenate([body, padv], axis=1).reshape(NW, NCH, CH)


def kernel(x, pos_edge_index, neg_edge_index, W1, b1, W2, b2, W3, b3):
    pos = pos_edge_index.astype(jnp.int32)
    neg = neg_edge_index.astype(jnp.int32)
    src = _pad_plan(jnp.concatenate([pos[0], pos[1]]), NSCR)
    dst = _pad_plan(jnp.concatenate([pos[1], pos[0]]), NSCR)
    sel0 = _pad_plan(jnp.concatenate([pos[0], neg[0]]), N)
    sel1 = _pad_plan(jnp.concatenate([pos[1], neg[1]]), N)

    cst = jnp.zeros((2 * CH, DG), jnp.float32).at[:CH, 0].set(1.0)
    dp = _deg_round(dst, cst)
    deg = dp[0, :, 0] + dp[1, :, 0] + 1.0
    real = jnp.arange(NPAD) < N
    dinv = jnp.where(real, lax.rsqrt(deg), 0.0)
    dinv2 = dinv * dinv

    x_pad = jnp.pad(x, ((0, NPAD - N), (0, 0)))
    xc, bc1, bc2 = _tc_prep(x_pad, W1, W2, W3, b1[None, :], b2[None, :])

    aug = jnp.concatenate(
        [xc, jnp.ones((NPAD, 1), jnp.float32),
         jnp.zeros((NPAD, AUG - D - 1), jnp.float32)], axis=1)
    zcst = jnp.zeros((CH, AUG), jnp.float32)
    s = dinv[:, None] * aug
    p = _round(s, src, dst, zcst)
    tot = p[0] + p[1] + s
    u1 = dinv * tot[:, D]
    s = dinv2[:, None] * tot
    p = _round(s, src, dst, zcst)
    tot = p[0] + p[1] + s
    u2 = dinv * tot[:, D]
    s = dinv2[:, None] * tot
    p = _round(s, src, dst, zcst)
    tot = p[0] + p[1] + s

    z = dinv[:, None] * tot[:, :D] + u2[:, None] * bc1[0] \
        + u1[:, None] * bc2[0] + b3[None, :]

    lp = _logits_call(z[:N], sel0, sel1)
    return lp.reshape(NW, WPE)[:, :EPW].reshape(-1)
